# baseline (speedup 1.0000x reference)
import functools

import jax
import jax.numpy as jnp
from jax import lax
from jax.experimental import pallas as pl
from jax.experimental.pallas import tpu as pltpu
from jax.experimental.pallas import tpu_sc as plsc

N_NODES = 10000
N_EDGES = 320000
D_FEAT = 128
N_ITERS = 3

NC = 2
NS = 16
NW = NC * NS
EH = N_EDGES // 2
EPW = EH // NW
CH = 1000
NCHUNK = EPW // CH

NB = 2000
NROW = N_NODES // 8
EROW = EH // 8
BE = 4000

_EPS = 1e-5


def _full(shape):
    return pl.BlockSpec(shape, lambda i: tuple(0 for _ in shape))


def _rows(shape):
    return pl.BlockSpec(shape, lambda i: (i,) + tuple(0 for _ in shape[1:]))


def _ln8(z, g, b):
    mu = jnp.mean(z, axis=-1, keepdims=True)
    var = jnp.mean((z - mu) ** 2, axis=-1, keepdims=True)
    return (z - mu) / jnp.sqrt(var + _EPS) * g + b


def _mm(a, b):
    return jnp.dot(a, b, preferred_element_type=jnp.float32)


def _init_body(x, wbig, bi, gi, bti, be1, bn1, we1a, we1b, wn1a, wn1b, wn1c,
               ts, te, c16, xpa, xqb, xc16):
    xw = _mm(x[...], wbig[...])
    h0 = jnp.tanh(_ln8(xw[:, 0:8] + bi[...], gi[...], bti[...]))
    xp = xw[:, 8:16] + be1[...]
    xq = xw[:, 16:24]
    xa = xw[:, 24:32]
    xb = xw[:, 32:40]
    xc = xw[:, 40:48] + bn1[...]
    zeros = jnp.zeros_like(xp)
    xpa[...] = jnp.concatenate([xp, xa], axis=-1)
    xqb[...] = jnp.concatenate([xq, xb], axis=-1)
    xc16[...] = jnp.concatenate([xc, zeros], axis=-1)
    p = _mm(h0, we1a[...]) + xp
    q = _mm(h0, we1b[...]) + xq
    a = _mm(h0, wn1a[...]) + xa
    b = _mm(h0, wn1b[...]) + xb
    ts[...] = jnp.concatenate([p, a], axis=-1)
    te[...] = jnp.concatenate([q, b], axis=-1)
    c16[...] = jnp.concatenate([_mm(h0, wn1c[...]) + xc, zeros], axis=-1)


def _tc_init(x, wbig, bi, gi, bti, be1, bn1, we1a, we1b, wn1a, wn1b, wn1c):
    f = jnp.float32
    n16 = jax.ShapeDtypeStruct((N_NODES, 16), f)
    return pl.pallas_call(
        _init_body,
        grid=(N_NODES // NB,),
        in_specs=[_rows((NB, D_FEAT)), _full((D_FEAT, 48))]
        + [_full((1, 8))] * 5
        + [_full((8, 8))] * 5,
        out_specs=[_rows((NB, 16))] * 6,
        out_shape=[n16] * 6,
    )(x, wbig, bi, gi, bti, be1, bn1, we1a, we1b, wn1a, wn1b, wn1c)


def _node_body(mia, moa, mib, mob, c16, xpa, xqb, xc16, ksw, mavg, wn2k, wn3k,
               wn4k, kpa, kqb, kc, lv, ts, te, cn):
    def lnt(s, i):
        mu = _mm(s, mavg[...])
        sc = s - mu
        var = _mm(sc * sc, mavg[...])
        return jnp.tanh(sc * jax.lax.rsqrt(var + _EPS) * lv[2 * i] + lv[2 * i + 1])

    acc = (mia[0] + mia[1] + moa[0] + moa[1]
           + mib[0] + mib[1] + mob[0] + mob[1])
    s = _mm(acc, ksw[...]) + c16[...]
    s = lnt(s, 0)
    s = lnt(_mm(s, wn2k[...]) + lv[8], 1)
    s = lnt(_mm(s, wn3k[...]) + lv[9], 2)
    s = lnt(_mm(s, wn4k[...]) + lv[10], 3)
    ts[...] = _mm(s, kpa[...]) + xpa[...]
    te[...] = _mm(s, kqb[...]) + xqb[...]
    cn[...] = _mm(s, kc[...]) + xc16[...]


def _tc_node(mia, moa, mib, mob, c16, xpa, xqb, xc16, nodew):
    f = jnp.float32
    n128 = jax.ShapeDtypeStruct((NROW, 128), f)
    return pl.pallas_call(
        _node_body,
        grid=(1,),
        in_specs=[_full((NC, NROW, 128))] * 4 + [_full((NROW, 128))] * 4
        + [_full((128, 128))] * 8 + [_full((11, 1, 128))],
        out_specs=[_full((NROW, 128))] * 3,
        out_shape=[n128] * 3,
    )(mia, moa, mib, mob, c16, xpa, xqb, xc16, *nodew)


def _edge_core(gs, ge, mavg, we2k, we3k, k4, lv):
    def lnt(s, i):
        mu = _mm(s, mavg[...])
        sc = s - mu
        var = _mm(sc * sc, mavg[...])
        return jnp.tanh(sc * jax.lax.rsqrt(var + _EPS) * lv[2 * i] + lv[2 * i + 1])

    s = lnt(gs + ge, 0)
    s = lnt(_mm(s, we2k[...]) + lv[6], 1)
    s = lnt(_mm(s, we3k[...]) + lv[7], 2)
    return _mm(s, k4[...])


def _edge_body(gs, ge, mavg, we2k, we3k, k4bc, lv, si, so):
    e = jax.nn.sigmoid(
        _edge_core(gs[...], ge[...], mavg, we2k, we3k, k4bc, lv) + lv[8])
    si[...] = e * gs[...]
    so[...] = e * ge[...]


def _edge_final_body(gs, ge, mavg, we2k, we3k, k4c8, lv, out):
    out[...] = (_edge_core(gs[...], ge[...], mavg, we2k, we3k, k4c8, lv)
                + lv[8][:, 0:8])


def _tc_edge(gs8, ge8, edgew):
    f = jnp.float32
    return pl.pallas_call(
        _edge_body,
        grid=(EROW // BE,),
        in_specs=[_rows((BE, 128))] * 2 + [_full((128, 128))] * 4
        + [_full((9, 1, 128))],
        out_specs=[_rows((BE, 128))] * 2,
        out_shape=[jax.ShapeDtypeStruct((EROW, 128), f)] * 2,
    )(gs8, ge8, *edgew)


def _tc_edge_final(gs8, ge8, edgew_final):
    return pl.pallas_call(
        _edge_final_body,
        grid=(EROW // BE,),
        in_specs=[_rows((BE, 128))] * 2 + [_full((128, 128))] * 3
        + [_full((128, 8))] + [_full((9, 1, 128))],
        out_specs=_rows((BE, 8)),
        out_shape=jax.ShapeDtypeStruct((EROW, 8), jnp.float32),
    )(gs8, ge8, *edgew_final)


_MESH = plsc.VectorSubcoreMesh(core_axis_name="c", subcore_axis_name="s",
                               num_cores=NC, num_subcores=NS)


@functools.partial(
    pl.kernel,
    out_type=[jax.ShapeDtypeStruct((EH, 16), jnp.float32),
              jax.ShapeDtypeStruct((EH, 16), jnp.float32)],
    mesh=_MESH,
    scratch_types=[pltpu.VMEM((EPW,), jnp.int32), pltpu.VMEM((EPW,), jnp.int32),
                   pltpu.VMEM((2, CH, 16), jnp.float32),
                   pltpu.VMEM((2, CH, 16), jnp.float32),
                   pltpu.VMEM_SHARED((N_NODES, 16), jnp.float32),
                   pltpu.VMEM_SHARED((N_NODES, 16), jnp.float32),
                   [pltpu.SemaphoreType.DMA] * 2, [pltpu.SemaphoreType.DMA] * 2,
                   [pltpu.SemaphoreType.DMA] * 2, [pltpu.SemaphoreType.DMA] * 2],
    compiler_params=pltpu.CompilerParams(use_tc_tiling_on_sc=False),
)
def _sc_gather(ts_hbm, te_hbm, s_hbm, e_hbm, gs_hbm, ge_hbm,
               idx_s, idx_e, rows1, rows2, sh_ts, sh_te,
               gsem1, gsem2, wsem1, wsem2):
    sid = lax.axis_index("s")
    wid = sid * NC + lax.axis_index("c")
    base = wid * EPW
    nsl = N_NODES // NS
    r0 = sid * nsl
    pltpu.sync_copy(ts_hbm.at[pl.ds(r0, nsl)], sh_ts.at[pl.ds(r0, nsl)])
    pltpu.sync_copy(te_hbm.at[pl.ds(r0, nsl)], sh_te.at[pl.ds(r0, nsl)])
    pltpu.sync_copy(s_hbm.at[pl.ds(base, EPW)], idx_s)
    pltpu.sync_copy(e_hbm.at[pl.ds(base, EPW)], idx_e)
    plsc.subcore_barrier()

    def start_gather(c):
        b = c % 2
        g1 = pltpu.async_copy(sh_ts.at[idx_s.at[pl.ds(c * CH, CH)]],
                              rows1.at[b], gsem1[b])
        g2 = pltpu.async_copy(sh_te.at[idx_e.at[pl.ds(c * CH, CH)]],
                              rows2.at[b], gsem2[b])
        return g1, g2

    pend_g = {0: start_gather(0)}
    pend_w = {}
    for c in range(NCHUNK):
        b = c % 2
        if c + 1 < NCHUNK:
            if c >= 1:
                for w in pend_w.pop(c - 1):
                    w.wait()
            pend_g[c + 1] = start_gather(c + 1)
        for g in pend_g.pop(c):
            g.wait()
        off = base + c * CH
        w1 = pltpu.async_copy(rows1.at[b], gs_hbm.at[pl.ds(off, CH)], wsem1[b])
        w2 = pltpu.async_copy(rows2.at[b], ge_hbm.at[pl.ds(off, CH)], wsem2[b])
        pend_w[c] = (w1, w2)
    for c in (NCHUNK - 2, NCHUNK - 1):
        for w in pend_w.pop(c, ()):
            w.wait()


@functools.partial(
    pl.kernel,
    out_type=[jax.ShapeDtypeStruct((NC, N_NODES, 16), jnp.float32),
              jax.ShapeDtypeStruct((NC, N_NODES, 16), jnp.float32)],
    mesh=_MESH,
    scratch_types=[pltpu.VMEM((CH,), jnp.int32), pltpu.VMEM((CH,), jnp.int32),
                   pltpu.VMEM((CH,), jnp.int32), pltpu.VMEM((CH,), jnp.int32),
                   pltpu.VMEM((2, CH, 16), jnp.float32),
                   pltpu.VMEM((2, CH, 16), jnp.float32),
                   pltpu.VMEM_SHARED((N_NODES, 16), jnp.float32),
                   pltpu.VMEM_SHARED((N_NODES, 16), jnp.float32),
                   [pltpu.SemaphoreType.DMA] * 2, [pltpu.SemaphoreType.DMA] * 2,
                   [pltpu.SemaphoreType.DMA] * 2, [pltpu.SemaphoreType.DMA] * 2],
    compiler_params=pltpu.CompilerParams(use_tc_tiling_on_sc=False),
)
def _sc_scatter(si_hbm, so_hbm, s_hbm, e_hbm, zz_hbm, mi_hbm, mo_hbm,
                idxi_a, idxi_b, idxo_a, idxo_b, rows1, rows2, acc_i, acc_o,
                lsem1, lsem2, ssem1, ssem2):
    cid = lax.axis_index("c")
    sid = lax.axis_index("s")
    wid = sid * NC + cid
    idxi = (idxi_a, idxi_b)
    idxo = (idxo_a, idxo_b)

    @pl.when(sid == 0)
    def _zero():
        pltpu.sync_copy(zz_hbm, acc_i)
        pltpu.sync_copy(zz_hbm, acc_o)

    plsc.subcore_barrier()
    base = wid * EPW

    def start_load(c):
        b = c % 2
        off = base + c * CH
        return (pltpu.async_copy(e_hbm.at[pl.ds(off, CH)], idxi[b], lsem1[b]),
                pltpu.async_copy(si_hbm.at[pl.ds(off, CH)], rows1.at[b], lsem1[b]),
                pltpu.async_copy(s_hbm.at[pl.ds(off, CH)], idxo[b], lsem2[b]),
                pltpu.async_copy(so_hbm.at[pl.ds(off, CH)], rows2.at[b], lsem2[b]))

    pend_l = {0: start_load(0)}
    pend_s = {}
    for c in range(NCHUNK):
        b = c % 2
        if c + 1 < NCHUNK:
            if c >= 1:
                for s in pend_s.pop(c - 1):
                    s.wait()
            pend_l[c + 1] = start_load(c + 1)
        for l in pend_l.pop(c):
            l.wait()
        s1 = pltpu.async_copy(rows1.at[b], acc_i.at[idxi[b]], ssem1[b], add=True)
        s2 = pltpu.async_copy(rows2.at[b], acc_o.at[idxo[b]], ssem2[b], add=True)
        pend_s[c] = (s1, s2)
    for c in (NCHUNK - 2, NCHUNK - 1):
        for s in pend_s.pop(c, ()):
            s.wait()
    plsc.subcore_barrier()

    @pl.when(sid == 0)
    def _flush():
        pltpu.sync_copy(acc_i, mi_hbm.at[cid])
        pltpu.sync_copy(acc_o, mo_hbm.at[cid])


def _blk16(m):
    return jnp.kron(jnp.eye(8, dtype=jnp.float32), m.astype(jnp.float32))


def _pad16(w8):
    z = jnp.zeros((8, 8), jnp.float32)
    return jnp.block([[w8.astype(jnp.float32), z], [z, z]])


def _lane(v8, hi=None):
    h = jnp.zeros((8,), jnp.float32) if hi is None else hi.astype(jnp.float32)
    return jnp.tile(jnp.concatenate([v8.astype(jnp.float32), h]), 8)[None, :]


def kernel(x, edge_index, input_params, edge_params, node_params):
    f = jnp.float32
    start = edge_index[0]
    end = edge_index[1]

    ip = input_params[0]
    we1 = edge_params[0]['W']
    wn1 = node_params[0]['W']

    wbig = jnp.concatenate(
        [ip['W'], we1[8:136], we1[144:272], wn1[8:136], wn1[144:272],
         wn1[280:408]], axis=1)

    def r18(v):
        return v.reshape(1, 8).astype(f)

    bi, gi, bti = r18(ip['b']), r18(ip['g']), r18(ip['beta'])
    be1 = r18(edge_params[0]['b'])
    bn1 = r18(node_params[0]['b'])
    we1a = we1[0:8]
    we1b = we1[136:144]
    wn1a = wn1[0:8]
    wn1b = wn1[136:144]
    wn1c = wn1[272:280]

    i8 = jnp.eye(8, dtype=f)
    z8 = jnp.zeros((8, 8), f)
    zv = jnp.zeros((8,), f)

    mavg = _blk16(_pad16(jnp.full((8, 8), 0.125, f)))
    ksw = _blk16(jnp.block([[z8, z8], [i8, z8]]))

    k4blk = jnp.zeros((16, 16), f).at[0:8, 0].set(edge_params[3]['W'][:, 0])
    bc16 = jnp.zeros((16, 16), f).at[0, :].set(1.0)
    k4bc = _blk16(k4blk @ bc16)
    kc8 = jnp.zeros((128, 8), f)
    kc8 = kc8.at[jnp.arange(8) * 16, jnp.arange(8)].set(1.0)
    k4c8 = _blk16(k4blk) @ kc8
    edge_lv = jnp.stack([
        _lane(edge_params[0]['g']), _lane(edge_params[0]['beta']),
        _lane(edge_params[1]['g']), _lane(edge_params[1]['beta']),
        _lane(edge_params[2]['g']), _lane(edge_params[2]['beta']),
        _lane(edge_params[1]['b']), _lane(edge_params[2]['b']),
        jnp.full((1, 128), edge_params[3]['b'][0], f),
    ])
    edgew = [mavg, _blk16(_pad16(edge_params[1]['W'])),
             _blk16(_pad16(edge_params[2]['W'])), k4bc, edge_lv]
    edgew_final = edgew[:3] + [k4c8, edge_lv]

    kpa = _blk16(jnp.block([[we1a.astype(f), wn1a.astype(f)], [z8, z8]]))
    kqb = _blk16(jnp.block([[we1b.astype(f), wn1b.astype(f)], [z8, z8]]))
    kc = _blk16(_pad16(wn1c))
    node_lv = jnp.stack([
        _lane(node_params[0]['g']), _lane(node_params[0]['beta']),
        _lane(node_params[1]['g']), _lane(node_params[1]['beta']),
        _lane(node_params[2]['g']), _lane(node_params[2]['beta']),
        _lane(node_params[3]['g']), _lane(node_params[3]['beta']),
        _lane(node_params[1]['b']), _lane(node_params[2]['b']),
        _lane(node_params[3]['b']),
    ])
    nodew = [ksw, mavg, _blk16(_pad16(node_params[1]['W'])),
             _blk16(_pad16(node_params[2]['W'])),
             _blk16(_pad16(node_params[3]['W'])), kpa, kqb, kc, node_lv]

    ts, te, c16, xpa, xqb, xc16 = _tc_init(
        x, wbig, bi, gi, bti, be1, bn1, we1a, we1b, wn1a, wn1b, wn1c)
    c16 = c16.reshape(NROW, 128)
    xpa = xpa.reshape(NROW, 128)
    xqb = xqb.reshape(NROW, 128)
    xc16 = xc16.reshape(NROW, 128)

    zz = jnp.zeros((N_NODES, 16), f)
    sa, sb = start[:EH], start[EH:]
    ea, eb = end[:EH], end[EH:]
    for _ in range(N_ITERS):
        gsa, gea = _sc_gather(ts, te, sa, ea)
        gsb, geb = _sc_gather(ts, te, sb, eb)
        sia, soa = _tc_edge(gsa.reshape(EROW, 128), gea.reshape(EROW, 128),
                            edgew)
        sib, sob = _tc_edge(gsb.reshape(EROW, 128), geb.reshape(EROW, 128),
                            edgew)
        mia, moa = _sc_scatter(sia.reshape(EH, 16), soa.reshape(EH, 16),
                               sa, ea, zz)
        mib, mob = _sc_scatter(sib.reshape(EH, 16), sob.reshape(EH, 16),
                               sb, eb, zz)
        ts, te, c16 = _tc_node(mia.reshape(NC, NROW, 128),
                               moa.reshape(NC, NROW, 128),
                               mib.reshape(NC, NROW, 128),
                               mob.reshape(NC, NROW, 128),
                               c16, xpa, xqb, xc16, nodew)
        ts = ts.reshape(N_NODES, 16)
        te = te.reshape(N_NODES, 16)

    gsa, gea = _sc_gather(ts, te, sa, ea)
    gsb, geb = _sc_gather(ts, te, sb, eb)
    la = _tc_edge_final(gsa.reshape(EROW, 128), gea.reshape(EROW, 128),
                        edgew_final)
    lb = _tc_edge_final(gsb.reshape(EROW, 128), geb.reshape(EROW, 128),
                        edgew_final)
    return jnp.concatenate([la, lb], axis=0).reshape(N_EDGES)


# speedup vs baseline: 1.0488x; 1.0488x over previous
import functools

import jax
import jax.numpy as jnp
from jax import lax
from jax.experimental import pallas as pl
from jax.experimental.pallas import tpu as pltpu
from jax.experimental.pallas import tpu_sc as plsc

N_NODES = 10000
N_EDGES = 320000
D_FEAT = 128
N_ITERS = 3

NC = 2
NS = 16
NW = NC * NS
EPW = N_EDGES // NW
CH = 1000
NCHUNK = EPW // CH

NB = 2000
NROW = N_NODES // 8
EROW = N_EDGES // 8
BE = 4000

_EPS = 1e-5


def _full(shape):
    return pl.BlockSpec(shape, lambda i: tuple(0 for _ in shape))


def _rows(shape):
    return pl.BlockSpec(shape, lambda i: (i,) + tuple(0 for _ in shape[1:]))


def _ln8(z, g, b):
    mu = jnp.mean(z, axis=-1, keepdims=True)
    var = jnp.mean((z - mu) ** 2, axis=-1, keepdims=True)
    return (z - mu) / jnp.sqrt(var + _EPS) * g + b


def _mm(a, b):
    return jnp.dot(a, b, preferred_element_type=jnp.float32)


def _init_body(x, wbig, bi, gi, bti, be1, bn1, we1a, we1b, wn1a, wn1b, wn1c,
               ts, te, c16, xpa, xqb, xc16):
    xw = _mm(x[...], wbig[...])
    h0 = jnp.tanh(_ln8(xw[:, 0:8] + bi[...], gi[...], bti[...]))
    xp = xw[:, 8:16] + be1[...]
    xq = xw[:, 16:24]
    xa = xw[:, 24:32]
    xb = xw[:, 32:40]
    xc = xw[:, 40:48] + bn1[...]
    zeros = jnp.zeros_like(xp)
    xpa[...] = jnp.concatenate([xp, xa], axis=-1)
    xqb[...] = jnp.concatenate([xq, xb], axis=-1)
    xc16[...] = jnp.concatenate([xc, zeros], axis=-1)
    p = _mm(h0, we1a[...]) + xp
    q = _mm(h0, we1b[...]) + xq
    a = _mm(h0, wn1a[...]) + xa
    b = _mm(h0, wn1b[...]) + xb
    ts[...] = jnp.concatenate([p, a], axis=-1)
    te[...] = jnp.concatenate([q, b], axis=-1)
    c16[...] = jnp.concatenate([_mm(h0, wn1c[...]) + xc, zeros], axis=-1)


def _tc_init(x, wbig, bi, gi, bti, be1, bn1, we1a, we1b, wn1a, wn1b, wn1c):
    f = jnp.float32
    n16 = jax.ShapeDtypeStruct((N_NODES, 16), f)
    return pl.pallas_call(
        _init_body,
        grid=(N_NODES // NB,),
        in_specs=[_rows((NB, D_FEAT)), _full((D_FEAT, 48))]
        + [_full((1, 8))] * 5
        + [_full((8, 8))] * 5,
        out_specs=[_rows((NB, 16))] * 6,
        out_shape=[n16] * 6,
    )(x, wbig, bi, gi, bti, be1, bn1, we1a, we1b, wn1a, wn1b, wn1c)


def _node_body(mi, mo, c16, xpa, xqb, xc16, ksw, mavg, wn2k, wn3k, wn4k,
               kpa, kqb, kc, lv, ts, te, cn):
    def lnt(s, i):
        mu = _mm(s, mavg[...])
        sc = s - mu
        var = _mm(sc * sc, mavg[...])
        return jnp.tanh(sc * jax.lax.rsqrt(var + _EPS) * lv[2 * i] + lv[2 * i + 1])

    acc = mi[0] + mi[1] + mo[0] + mo[1]
    s = _mm(acc, ksw[...]) + c16[...]
    s = lnt(s, 0)
    s = lnt(_mm(s, wn2k[...]) + lv[8], 1)
    s = lnt(_mm(s, wn3k[...]) + lv[9], 2)
    s = lnt(_mm(s, wn4k[...]) + lv[10], 3)
    ts[...] = _mm(s, kpa[...]) + xpa[...]
    te[...] = _mm(s, kqb[...]) + xqb[...]
    cn[...] = _mm(s, kc[...]) + xc16[...]


def _tc_node(mi_p, mo_p, c16, xpa, xqb, xc16, nodew):
    f = jnp.float32
    n128 = jax.ShapeDtypeStruct((NROW, 128), f)
    return pl.pallas_call(
        _node_body,
        grid=(1,),
        in_specs=[_full((NC, NROW, 128))] * 2 + [_full((NROW, 128))] * 4
        + [_full((128, 128))] * 8 + [_full((11, 1, 128))],
        out_specs=[_full((NROW, 128))] * 3,
        out_shape=[n128] * 3,
    )(mi_p, mo_p, c16, xpa, xqb, xc16, *nodew)


def _edge_core(gs, ge, mavg, we2k, we3k, k4, lv):
    def lnt(s, i):
        mu = _mm(s, mavg[...])
        sc = s - mu
        var = _mm(sc * sc, mavg[...])
        return jnp.tanh(sc * jax.lax.rsqrt(var + _EPS) * lv[2 * i] + lv[2 * i + 1])

    s = lnt(gs + ge, 0)
    s = lnt(_mm(s, we2k[...]) + lv[6], 1)
    s = lnt(_mm(s, we3k[...]) + lv[7], 2)
    return _mm(s, k4[...])


def _edge_body(gs, ge, mavg, we2k, we3k, k4bc, lv, si, so):
    e = jax.nn.sigmoid(
        _edge_core(gs[...], ge[...], mavg, we2k, we3k, k4bc, lv) + lv[8])
    si[...] = e * gs[...]
    so[...] = e * ge[...]


def _edge_final_body(gs, ge, mavg, we2k, we3k, k4c8, lv, out):
    out[...] = (_edge_core(gs[...], ge[...], mavg, we2k, we3k, k4c8, lv)
                + lv[8][:, 0:8])


def _tc_edge(gs8, ge8, edgew):
    f = jnp.float32
    return pl.pallas_call(
        _edge_body,
        grid=(EROW // BE,),
        in_specs=[_rows((BE, 128))] * 2 + [_full((128, 128))] * 4
        + [_full((9, 1, 128))],
        out_specs=[_rows((BE, 128))] * 2,
        out_shape=[jax.ShapeDtypeStruct((EROW, 128), f)] * 2,
    )(gs8, ge8, *edgew)


def _tc_edge_final(gs8, ge8, edgew_final):
    return pl.pallas_call(
        _edge_final_body,
        grid=(EROW // BE,),
        in_specs=[_rows((BE, 128))] * 2 + [_full((128, 128))] * 3
        + [_full((128, 8))] + [_full((9, 1, 128))],
        out_specs=_rows((BE, 8)),
        out_shape=jax.ShapeDtypeStruct((EROW, 8), jnp.float32),
    )(gs8, ge8, *edgew_final)


_MESH = plsc.VectorSubcoreMesh(core_axis_name="c", subcore_axis_name="s",
                               num_cores=NC, num_subcores=NS)


@functools.partial(
    pl.kernel,
    out_type=[jax.ShapeDtypeStruct((N_EDGES, 16), jnp.float32),
              jax.ShapeDtypeStruct((N_EDGES, 16), jnp.float32)],
    mesh=_MESH,
    scratch_types=[pltpu.VMEM((EPW,), jnp.int32), pltpu.VMEM((EPW,), jnp.int32),
                   pltpu.VMEM((2, CH, 16), jnp.float32),
                   pltpu.VMEM((2, CH, 16), jnp.float32),
                   pltpu.VMEM_SHARED((N_NODES, 16), jnp.float32),
                   pltpu.VMEM_SHARED((N_NODES, 16), jnp.float32),
                   [pltpu.SemaphoreType.DMA] * 2, [pltpu.SemaphoreType.DMA] * 2,
                   [pltpu.SemaphoreType.DMA] * 2, [pltpu.SemaphoreType.DMA] * 2],
    compiler_params=pltpu.CompilerParams(use_tc_tiling_on_sc=False),
)
def _sc_gather(ts_hbm, te_hbm, s_hbm, e_hbm, gs_hbm, ge_hbm,
               idx_s, idx_e, rows1, rows2, sh_ts, sh_te,
               gsem1, gsem2, wsem1, wsem2):
    sid = lax.axis_index("s")
    wid = sid * NC + lax.axis_index("c")
    base = wid * EPW
    nsl = N_NODES // NS
    r0 = sid * nsl
    pltpu.sync_copy(ts_hbm.at[pl.ds(r0, nsl)], sh_ts.at[pl.ds(r0, nsl)])
    pltpu.sync_copy(te_hbm.at[pl.ds(r0, nsl)], sh_te.at[pl.ds(r0, nsl)])
    pltpu.sync_copy(s_hbm.at[pl.ds(base, EPW)], idx_s)
    pltpu.sync_copy(e_hbm.at[pl.ds(base, EPW)], idx_e)
    plsc.subcore_barrier()

    def start_gather(c):
        b = c % 2
        g1 = pltpu.async_copy(sh_ts.at[idx_s.at[pl.ds(c * CH, CH)]],
                              rows1.at[b], gsem1[b])
        g2 = pltpu.async_copy(sh_te.at[idx_e.at[pl.ds(c * CH, CH)]],
                              rows2.at[b], gsem2[b])
        return g1, g2

    pend_g = {0: start_gather(0)}
    pend_w = {}
    for c in range(NCHUNK):
        b = c % 2
        if c + 1 < NCHUNK:
            if c >= 1:
                for w in pend_w.pop(c - 1):
                    w.wait()
            pend_g[c + 1] = start_gather(c + 1)
        for g in pend_g.pop(c):
            g.wait()
        off = base + c * CH
        w1 = pltpu.async_copy(rows1.at[b], gs_hbm.at[pl.ds(off, CH)], wsem1[b])
        w2 = pltpu.async_copy(rows2.at[b], ge_hbm.at[pl.ds(off, CH)], wsem2[b])
        pend_w[c] = (w1, w2)
    for c in (NCHUNK - 2, NCHUNK - 1):
        for w in pend_w.pop(c, ()):
            w.wait()


@functools.partial(
    pl.kernel,
    out_type=[jax.ShapeDtypeStruct((NC, N_NODES, 16), jnp.float32),
              jax.ShapeDtypeStruct((NC, N_NODES, 16), jnp.float32)],
    mesh=_MESH,
    scratch_types=[pltpu.VMEM((CH,), jnp.int32), pltpu.VMEM((CH,), jnp.int32),
                   pltpu.VMEM((CH,), jnp.int32), pltpu.VMEM((CH,), jnp.int32),
                   pltpu.VMEM((2, CH, 16), jnp.float32),
                   pltpu.VMEM((2, CH, 16), jnp.float32),
                   pltpu.VMEM_SHARED((N_NODES, 16), jnp.float32),
                   pltpu.VMEM_SHARED((N_NODES, 16), jnp.float32),
                   [pltpu.SemaphoreType.DMA] * 2, [pltpu.SemaphoreType.DMA] * 2,
                   [pltpu.SemaphoreType.DMA] * 2, [pltpu.SemaphoreType.DMA] * 2],
    compiler_params=pltpu.CompilerParams(use_tc_tiling_on_sc=False),
)
def _sc_scatter(si_hbm, so_hbm, s_hbm, e_hbm, zz_hbm, mi_hbm, mo_hbm,
                idxi_a, idxi_b, idxo_a, idxo_b, rows1, rows2, acc_i, acc_o,
                lsem1, lsem2, ssem1, ssem2):
    cid = lax.axis_index("c")
    sid = lax.axis_index("s")
    wid = sid * NC + cid
    idxi = (idxi_a, idxi_b)
    idxo = (idxo_a, idxo_b)

    @pl.when(sid == 0)
    def _zero():
        pltpu.sync_copy(zz_hbm, acc_i)
        pltpu.sync_copy(zz_hbm, acc_o)

    plsc.subcore_barrier()
    base = wid * EPW

    def start_load(c):
        b = c % 2
        off = base + c * CH
        return (pltpu.async_copy(e_hbm.at[pl.ds(off, CH)], idxi[b], lsem1[b]),
                pltpu.async_copy(si_hbm.at[pl.ds(off, CH)], rows1.at[b], lsem1[b]),
                pltpu.async_copy(s_hbm.at[pl.ds(off, CH)], idxo[b], lsem2[b]),
                pltpu.async_copy(so_hbm.at[pl.ds(off, CH)], rows2.at[b], lsem2[b]))

    pend_l = {0: start_load(0)}
    pend_s = {}
    for c in range(NCHUNK):
        b = c % 2
        if c + 1 < NCHUNK:
            if c >= 1:
                for s in pend_s.pop(c - 1):
                    s.wait()
            pend_l[c + 1] = start_load(c + 1)
        for l in pend_l.pop(c):
            l.wait()
        s1 = pltpu.async_copy(rows1.at[b], acc_i.at[idxi[b]], ssem1[b], add=True)
        s2 = pltpu.async_copy(rows2.at[b], acc_o.at[idxo[b]], ssem2[b], add=True)
        pend_s[c] = (s1, s2)
    for c in (NCHUNK - 2, NCHUNK - 1):
        for s in pend_s.pop(c, ()):
            s.wait()
    plsc.subcore_barrier()

    @pl.when(sid == 0)
    def _flush():
        pltpu.sync_copy(acc_i, mi_hbm.at[cid])
        pltpu.sync_copy(acc_o, mo_hbm.at[cid])


def _blk16(m):
    return jnp.kron(jnp.eye(8, dtype=jnp.float32), m.astype(jnp.float32))


def _pad16(w8):
    z = jnp.zeros((8, 8), jnp.float32)
    return jnp.block([[w8.astype(jnp.float32), z], [z, z]])


def _lane(v8, hi=None):
    h = jnp.zeros((8,), jnp.float32) if hi is None else hi.astype(jnp.float32)
    return jnp.tile(jnp.concatenate([v8.astype(jnp.float32), h]), 8)[None, :]


def kernel(x, edge_index, input_params, edge_params, node_params):
    f = jnp.float32
    start = edge_index[0]
    end = edge_index[1]

    ip = input_params[0]
    we1 = edge_params[0]['W']
    wn1 = node_params[0]['W']

    wbig = jnp.concatenate(
        [ip['W'], we1[8:136], we1[144:272], wn1[8:136], wn1[144:272],
         wn1[280:408]], axis=1)

    def r18(v):
        return v.reshape(1, 8).astype(f)

    bi, gi, bti = r18(ip['b']), r18(ip['g']), r18(ip['beta'])
    be1 = r18(edge_params[0]['b'])
    bn1 = r18(node_params[0]['b'])
    we1a = we1[0:8]
    we1b = we1[136:144]
    wn1a = wn1[0:8]
    wn1b = wn1[136:144]
    wn1c = wn1[272:280]

    i8 = jnp.eye(8, dtype=f)
    z8 = jnp.zeros((8, 8), f)
    zv = jnp.zeros((8,), f)

    mavg = _blk16(_pad16(jnp.full((8, 8), 0.125, f)))
    ksw = _blk16(jnp.block([[z8, z8], [i8, z8]]))

    k4blk = jnp.zeros((16, 16), f).at[0:8, 0].set(edge_params[3]['W'][:, 0])
    bc16 = jnp.zeros((16, 16), f).at[0, :].set(1.0)
    k4bc = _blk16(k4blk @ bc16)
    kc8 = jnp.zeros((128, 8), f)
    kc8 = kc8.at[jnp.arange(8) * 16, jnp.arange(8)].set(1.0)
    k4c8 = _blk16(k4blk) @ kc8
    edge_lv = jnp.stack([
        _lane(edge_params[0]['g']), _lane(edge_params[0]['beta']),
        _lane(edge_params[1]['g']), _lane(edge_params[1]['beta']),
        _lane(edge_params[2]['g']), _lane(edge_params[2]['beta']),
        _lane(edge_params[1]['b']), _lane(edge_params[2]['b']),
        jnp.full((1, 128), edge_params[3]['b'][0], f),
    ])
    edgew = [mavg, _blk16(_pad16(edge_params[1]['W'])),
             _blk16(_pad16(edge_params[2]['W'])), k4bc, edge_lv]
    edgew_final = edgew[:3] + [k4c8, edge_lv]

    kpa = _blk16(jnp.block([[we1a.astype(f), wn1a.astype(f)], [z8, z8]]))
    kqb = _blk16(jnp.block([[we1b.astype(f), wn1b.astype(f)], [z8, z8]]))
    kc = _blk16(_pad16(wn1c))
    node_lv = jnp.stack([
        _lane(node_params[0]['g']), _lane(node_params[0]['beta']),
        _lane(node_params[1]['g']), _lane(node_params[1]['beta']),
        _lane(node_params[2]['g']), _lane(node_params[2]['beta']),
        _lane(node_params[3]['g']), _lane(node_params[3]['beta']),
        _lane(node_params[1]['b']), _lane(node_params[2]['b']),
        _lane(node_params[3]['b']),
    ])
    nodew = [ksw, mavg, _blk16(_pad16(node_params[1]['W'])),
             _blk16(_pad16(node_params[2]['W'])),
             _blk16(_pad16(node_params[3]['W'])), kpa, kqb, kc, node_lv]

    ts, te, c16, xpa, xqb, xc16 = _tc_init(
        x, wbig, bi, gi, bti, be1, bn1, we1a, we1b, wn1a, wn1b, wn1c)
    c16 = c16.reshape(NROW, 128)
    xpa = xpa.reshape(NROW, 128)
    xqb = xqb.reshape(NROW, 128)
    xc16 = xc16.reshape(NROW, 128)

    zz = jnp.zeros((N_NODES, 16), f)
    for _ in range(N_ITERS):
        gs, ge = _sc_gather(ts, te, start, end)
        si, so = _tc_edge(gs.reshape(EROW, 128), ge.reshape(EROW, 128), edgew)
        mi_p, mo_p = _sc_scatter(si.reshape(N_EDGES, 16),
                                 so.reshape(N_EDGES, 16), start, end, zz)
        ts, te, c16 = _tc_node(mi_p.reshape(NC, NROW, 128),
                               mo_p.reshape(NC, NROW, 128),
                               c16, xpa, xqb, xc16, nodew)
        ts = ts.reshape(N_NODES, 16)
        te = te.reshape(N_NODES, 16)

    gs, ge = _sc_gather(ts, te, start, end)
    logits = _tc_edge_final(gs.reshape(EROW, 128), ge.reshape(EROW, 128),
                            edgew_final)
    return logits.reshape(N_EDGES)
